# baseline (speedup 1.0000x reference)
import functools

import jax
import jax.numpy as jnp
from jax import lax
from jax.experimental import pallas as pl
from jax.experimental.pallas import tpu as pltpu
from jax.experimental.pallas import tpu_sc as plsc

N = 10000
E = 320000
D = 128
G = 64
NPAD = 10240
NC, NS, L = 2, 16, 16
EPC = E // NC
EPT = EPC // NS
EBLK = 400
NBLK = EPT // EBLK
NCHUNK = EBLK // L
RING = 6
RPT = NPAD // NS
BPW = NPAD // (NC * NS)
BLK = 1024
GRID = NPAD // BLK

_mesh = plsc.VectorSubcoreMesh(core_axis_name="c", subcore_axis_name="s")


@functools.partial(
    pl.kernel,
    out_type=jax.ShapeDtypeStruct((NPAD, D), jnp.float32),
    mesh=_mesh,
    scratch_types=[
        pltpu.VMEM((BPW,), jnp.int32),
        pltpu.VMEM((BPW, D), jnp.float32),
        pltpu.SemaphoreType.DMA,
    ],
)
def _emb_gather(emb_hbm, idx_hbm, out_hbm, idx_v, rows_v, sem):
    w = lax.axis_index("s") * NC + lax.axis_index("c")
    base = w * BPW
    pltpu.sync_copy(idx_hbm.at[pl.ds(base, BPW)], idx_v)
    pltpu.async_copy(emb_hbm.at[idx_v], rows_v, sem).wait()
    pltpu.sync_copy(rows_v, out_hbm.at[pl.ds(base, BPW)])


@functools.partial(
    pl.kernel,
    out_type=(jax.ShapeDtypeStruct((NC, NPAD, D), jnp.float32),
              jax.ShapeDtypeStruct((NC, NS, NPAD), jnp.float32)),
    mesh=_mesh,
    compiler_params=pltpu.CompilerParams(needs_layout_passes=False),
    scratch_types=[
        pltpu.VMEM((EBLK,), jnp.int32),
        pltpu.VMEM((EBLK,), jnp.int32),
        pltpu.VMEM((EBLK,), jnp.float32),
        pltpu.VMEM((NPAD,), jnp.float32),
        pltpu.VMEM((NPAD,), jnp.float32),
        pltpu.VMEM((NPAD,), jnp.float32),
    pltpu.VMEM((L,), jnp.float32),
    ] + [pltpu.VMEM((L, D), jnp.float32) for _ in range(RING)]
      + [pltpu.VMEM((L,), jnp.int32) for _ in range(2 * RING)]
      + [pltpu.VMEM_SHARED((NPAD, D), jnp.float32)]
      + [pltpu.SemaphoreType.DMA for _ in range(2 * RING)],
)
def _edge(h_hbm, p_hbm, q_hbm, src_hbm, dst_hbm, ew_hbm,
          agg_hbm, s_hbm,
          src_v, dst_v, ew_v, p_v, q_v, s_loc, tbuf, *rest):
    rows = list(rest[0:RING])
    ibufs = list(rest[RING:2 * RING])
    dbufs = list(rest[2 * RING:3 * RING])
    agg_s = rest[3 * RING]
    gsems = list(rest[3 * RING + 1:4 * RING + 1])
    ssems = list(rest[4 * RING + 1:5 * RING + 1])
    c = lax.axis_index("c")
    t = lax.axis_index("s")
    ebase = c * EPC + t * EPT
    pltpu.sync_copy(p_hbm, p_v)
    pltpu.sync_copy(q_hbm, q_v)

    zv = jnp.zeros((L,), jnp.float32)

    def _zs(i, carry):
        s_loc[pl.ds(i * L, L)] = zv
        return carry
    lax.fori_loop(0, NPAD // L, _zs, 0)

    for r in range(L):
        for k in range(D // L):
            rows[0][r, pl.ds(k * L, L)] = zv

    rbase = t * RPT

    def _za(j, carry):
        pltpu.sync_copy(rows[0], agg_s.at[pl.ds(rbase + j * L, L)])
        return carry
    lax.fori_loop(0, RPT // L, _za, 0)
    plsc.subcore_barrier()

    def _scale_scatter(rows, dbuf, ssem, tv, dv):
        dbuf[...] = dv
        for r in range(L):
            tr = tv[r]
            for k in range(D // L):
                rows[r, pl.ds(k * L, L)] = rows[r, pl.ds(k * L, L)] * tr
        pltpu.async_copy(rows, agg_s.at[dbuf], ssem, add=True)


    def _blk(bi, carry):
        eb = ebase + bi * EBLK
        pltpu.sync_copy(src_hbm.at[pl.ds(eb, EBLK)], src_v)
        pltpu.sync_copy(dst_hbm.at[pl.ds(eb, EBLK)], dst_v)
        pltpu.sync_copy(ew_hbm.at[pl.ds(eb, EBLK)], ew_v)

        for j in range(RING - 1):
            ibufs[j][...] = src_v[pl.ds(j * L, L)]
            pltpu.async_copy(h_hbm.at[ibufs[j]], rows[j], gsems[j])

        def _body(ci, carry2):
            par = lax.rem(ci, RING)
            off = ci * L
            sv = src_v[pl.ds(off, L)]
            dv = dst_v[pl.ds(off, L)]

            @pl.when(ci < NCHUNK - (RING - 1))
            def _():
                nsv = src_v[pl.ds(off + (RING - 1) * L, L)]
                npar = lax.rem(ci + RING - 1, RING)
                for j in range(RING):
                    @pl.when(npar == j)
                    def _(j=j):
                        @pl.when(ci >= 1)
                        def _():
                            pltpu.make_async_copy(
                                rows[j], agg_s.at[dbufs[j]], ssems[j]).wait()
                        ibufs[j][...] = nsv
                        pltpu.async_copy(h_hbm.at[ibufs[j]], rows[j],
                                         gsems[j])

            pj = plsc.load_gather(p_v, [dv])
            qj = plsc.load_gather(q_v, [sv])
            a = pj + qj
            a = jnp.where(a > 0, a, 0.2 * a)
            e = jnp.exp(a)
            plsc.addupdate_scatter(s_loc, [dv], e)
            tbuf[...] = e * ew_v[pl.ds(off, L)]
            tv = tbuf[...]

            for j in range(RING):
                @pl.when(par == j)
                def _(j=j):
                    pltpu.make_async_copy(
                        h_hbm.at[ibufs[j]], rows[j], gsems[j]).wait()
                    _scale_scatter(rows[j], dbufs[j], ssems[j], tv, dv)
            return carry2
        lax.fori_loop(0, NCHUNK, _body, 0)
        for j in range(RING):
            pltpu.make_async_copy(rows[j], agg_s.at[dbufs[j]],
                                  ssems[j]).wait()
        return carry
    lax.fori_loop(0, NBLK, _blk, 0)
    plsc.subcore_barrier()

    pltpu.sync_copy(s_loc, s_hbm.at[c, t])
    pltpu.sync_copy(agg_s.at[pl.ds(rbase, RPT)],
                    agg_hbm.at[c, pl.ds(rbase, RPT)])


def _ln_attn_body(h_ref, ls_ref, lb_ref, awd_ref, aws_ref,
                  hln_ref, p_ref, q_ref):
    xv = h_ref[...]
    mu = jnp.mean(xv, axis=-1, keepdims=True)
    var = jnp.mean(jnp.square(xv - mu), axis=-1, keepdims=True)
    hln = (xv - mu) / jnp.sqrt(var + 1e-5) * ls_ref[...] + lb_ref[...]
    hln_ref[...] = hln
    p_ref[...] = jnp.sum(hln * awd_ref[...], axis=-1, keepdims=True)
    q_ref[...] = jnp.sum(hln * aws_ref[...], axis=-1, keepdims=True)


def _ln_attn(h, ls, lb, awd, aws):
    row = pl.BlockSpec((BLK, D), lambda i: (i, 0))
    vec = pl.BlockSpec((1, D), lambda i: (0, 0))
    col = pl.BlockSpec((BLK, 1), lambda i: (i, 0))
    return pl.pallas_call(
        _ln_attn_body,
        grid=(GRID,),
        in_specs=[row, vec, vec, vec, vec],
        out_specs=[row, col, col],
        out_shape=[
            jax.ShapeDtypeStruct((NPAD, D), jnp.float32),
            jax.ShapeDtypeStruct((NPAD, 1), jnp.float32),
            jax.ShapeDtypeStruct((NPAD, 1), jnp.float32),
        ],
    )(h, ls, lb, awd, aws)


def _make_mlp_body(do_ln):
    def body(h_ref, a0_ref, a1_ref, s_ref, p_ref, q_ref, ep_ref,
             w1_ref, b1_ref, w2_ref, b2_ref, *rest):
        hln = h_ref[...]
        es = p_ref[...] + q_ref[...]
        es = jnp.where(es > 0, es, 0.2 * es)
        es = jnp.exp(es)
        stot = jnp.sum(s_ref[...], axis=-1, keepdims=True) + es
        aggf = (a0_ref[...] + a1_ref[...] + es * hln) / stot
        outv = ep_ref[...][:1, :1] * hln + aggf
        h1 = jnp.dot(outv, w1_ref[...],
                     preferred_element_type=jnp.float32) + b1_ref[...]
        h1 = jnp.maximum(h1, 0.0)
        y = jnp.dot(h1, w2_ref[...],
                    preferred_element_type=jnp.float32) + b2_ref[...]
        y = jnp.maximum(y, 0.0)
        if do_ln:
            ls_ref, lb_ref, awd_ref, aws_ref, hln_ref, p2_ref, q2_ref = rest
            mu = jnp.mean(y, axis=-1, keepdims=True)
            var = jnp.mean(jnp.square(y - mu), axis=-1, keepdims=True)
            y2 = (y - mu) / jnp.sqrt(var + 1e-5) * ls_ref[...] + lb_ref[...]
            hln_ref[...] = y2
            p2_ref[...] = jnp.sum(y2 * awd_ref[...], axis=-1, keepdims=True)
            q2_ref[...] = jnp.sum(y2 * aws_ref[...], axis=-1, keepdims=True)
        else:
            (y_ref,) = rest
            y_ref[...] = y
    return body


def _mlp(do_ln, h, a0, a1, st, p, q, ep, w1, b1, w2, b2, extra):
    row = pl.BlockSpec((BLK, D), lambda i: (i, 0))
    vec = pl.BlockSpec((1, D), lambda i: (0, 0))
    col = pl.BlockSpec((BLK, 1), lambda i: (i, 0))
    mat = pl.BlockSpec((D, D), lambda i: (0, 0))
    srow = pl.BlockSpec((BLK, NC * NS), lambda i: (i, 0))
    in_specs = [row, row, row, srow, col, col, vec, mat, vec, mat, vec]
    args = [h, a0, a1, st, p, q, ep, w1, b1, w2, b2]
    if do_ln:
        in_specs += [vec, vec, vec, vec]
        out_specs = [row, col, col]
        out_shape = [
            jax.ShapeDtypeStruct((NPAD, D), jnp.float32),
            jax.ShapeDtypeStruct((NPAD, 1), jnp.float32),
            jax.ShapeDtypeStruct((NPAD, 1), jnp.float32),
        ]
    else:
        out_specs = [row]
        out_shape = [jax.ShapeDtypeStruct((NPAD, D), jnp.float32)]
    return pl.pallas_call(
        _make_mlp_body(do_ln),
        grid=(GRID,),
        in_specs=in_specs,
        out_specs=out_specs,
        out_shape=out_shape,
    )(*args, *extra)


def _pool_body(y_ref, b_ref, w3_ref, b3_ref, w4_ref, b4_ref,
               out_ref, sums, cnts):
    i = pl.program_id(0)

    @pl.when(i == 0)
    def _():
        sums[...] = jnp.zeros((G, D), jnp.float32)
        cnts[...] = jnp.zeros((G, D), jnp.float32)

    bb = b_ref[...]
    oh = (lax.broadcasted_iota(jnp.int32, (G, BLK), 0) == bb
          ).astype(jnp.float32)
    sums[...] += jnp.dot(oh, y_ref[...], preferred_element_type=jnp.float32)
    cnts[...] += jnp.broadcast_to(
        jnp.sum(oh, axis=-1, keepdims=True), (G, D))

    @pl.when(i == GRID - 1)
    def _():
        g = sums[...] / jnp.maximum(cnts[...], 1.0)
        z = jnp.maximum(
            jnp.dot(g, w3_ref[...], preferred_element_type=jnp.float32)
            + b3_ref[...], 0.0)
        out_ref[...] = jnp.dot(
            z, w4_ref[...], preferred_element_type=jnp.float32) + b4_ref[...]


def _pool(y, bpad, w3, b3, w4, b4):
    return pl.pallas_call(
        _pool_body,
        grid=(GRID,),
        in_specs=[
            pl.BlockSpec((BLK, D), lambda i: (i, 0)),
            pl.BlockSpec((1, BLK), lambda i: (0, i)),
            pl.BlockSpec((D, G), lambda i: (0, 0)),
            pl.BlockSpec((1, G), lambda i: (0, 0)),
            pl.BlockSpec((G, 2), lambda i: (0, 0)),
            pl.BlockSpec((1, 2), lambda i: (0, 0)),
        ],
        out_specs=pl.BlockSpec((G, 2), lambda i: (0, 0)),
        out_shape=jax.ShapeDtypeStruct((G, 2), jnp.float32),
        scratch_shapes=[
            pltpu.VMEM((G, D), jnp.float32),
            pltpu.VMEM((G, D), jnp.float32),
        ],
        compiler_params=pltpu.CompilerParams(
            dimension_semantics=("arbitrary",)),
    )(y, bpad, w3, b3, w4, b4)


def kernel(x, edge_index, edge_attr, batch, emb,
           ln0_s, ln0_b, att0, w1_0, b1_0, w2_0, b2_0, eps0,
           ln1_s, ln1_b, att1, w1_1, b1_1, w2_1, b2_1, eps1,
           w3, b3, w4, b4):
    f32 = jnp.float32
    xi = jnp.concatenate([x.astype(jnp.int32),
                          jnp.zeros((NPAD - N,), jnp.int32)])
    src = edge_index[0].astype(jnp.int32)
    dst = edge_index[1].astype(jnp.int32)
    ew = edge_attr.astype(f32)
    bpad = jnp.concatenate([batch.astype(jnp.int32),
                            jnp.full((NPAD - N,), G, jnp.int32)]
                           ).reshape(1, NPAD)

    awd0 = att0[:D, 0].reshape(1, D)
    aws0 = att0[D:, 0].reshape(1, D)
    awd1 = att1[:D, 0].reshape(1, D)
    aws1 = att1[D:, 0].reshape(1, D)

    h0 = _emb_gather(emb.astype(f32), xi)
    hln1, p1, q1 = _ln_attn(h0, ln0_s.reshape(1, D), ln0_b.reshape(1, D),
                            awd0, aws0)
    agg1, s1 = _edge(hln1, p1.reshape(NPAD), q1.reshape(NPAD), src, dst, ew)
    s1t = jnp.swapaxes(s1.reshape(NC * NS, NPAD), 0, 1)
    ep0 = jnp.broadcast_to(jnp.reshape(1.0 + eps0, (1, 1)), (1, D))
    hln2, p2, q2 = _mlp(True, hln1, agg1[0], agg1[1], s1t, p1, q1, ep0,
                        w1_0, b1_0.reshape(1, D), w2_0, b2_0.reshape(1, D),
                        [ln1_s.reshape(1, D), ln1_b.reshape(1, D),
                         awd1, aws1])
    agg2, s2 = _edge(hln2, p2.reshape(NPAD), q2.reshape(NPAD), src, dst, ew)
    s2t = jnp.swapaxes(s2.reshape(NC * NS, NPAD), 0, 1)
    ep1 = jnp.broadcast_to(jnp.reshape(1.0 + eps1, (1, 1)), (1, D))
    (y2,) = _mlp(False, hln2, agg2[0], agg2[1], s2t, p2, q2, ep1,
                 w1_1, b1_1.reshape(1, D), w2_1, b2_1.reshape(1, D), [])
    return _pool(y2, bpad, w3, b3.reshape(1, G), w4, b4.reshape(1, 2))


# speedup vs baseline: 1.1474x; 1.1474x over previous
import functools

import jax
import jax.numpy as jnp
from jax import lax
from jax.experimental import pallas as pl
from jax.experimental.pallas import tpu as pltpu
from jax.experimental.pallas import tpu_sc as plsc

N = 10000
E = 320000
D = 128
G = 64
NPAD = 10240
NC, NS, L = 2, 16, 16
EPC = E // NC
EPT = EPC // NS
EBLK = 2000
NBLK = EPT // EBLK
NCHUNK = EBLK // L
RING = 5
RPT = NPAD // NS
BPW = NPAD // (NC * NS)
BLK = 1024
GRID = NPAD // BLK

_mesh = plsc.VectorSubcoreMesh(core_axis_name="c", subcore_axis_name="s")


@functools.partial(
    pl.kernel,
    out_type=jax.ShapeDtypeStruct((NPAD, D), jnp.float32),
    mesh=_mesh,
    scratch_types=[
        pltpu.VMEM((BPW,), jnp.int32),
        pltpu.VMEM((BPW, D), jnp.float32),
        pltpu.SemaphoreType.DMA,
    ],
)
def _emb_gather(emb_hbm, idx_hbm, out_hbm, idx_v, rows_v, sem):
    w = lax.axis_index("s") * NC + lax.axis_index("c")
    base = w * BPW
    pltpu.sync_copy(idx_hbm.at[pl.ds(base, BPW)], idx_v)
    pltpu.async_copy(emb_hbm.at[idx_v], rows_v, sem).wait()
    pltpu.sync_copy(rows_v, out_hbm.at[pl.ds(base, BPW)])


@functools.partial(
    pl.kernel,
    out_type=(jax.ShapeDtypeStruct((NC, NPAD, D), jnp.float32),
              jax.ShapeDtypeStruct((NC, NS, NPAD), jnp.float32)),
    mesh=_mesh,
    compiler_params=pltpu.CompilerParams(needs_layout_passes=False),
    scratch_types=[
        pltpu.VMEM((EBLK,), jnp.int32),
        pltpu.VMEM((EBLK,), jnp.int32),
        pltpu.VMEM((EBLK,), jnp.float32),
        pltpu.VMEM((NPAD,), jnp.float32),
        pltpu.VMEM((NPAD,), jnp.float32),
        pltpu.VMEM((NPAD,), jnp.float32),
    pltpu.VMEM((L,), jnp.float32),
    ] + [pltpu.VMEM((L, D), jnp.float32) for _ in range(RING)]
      + [pltpu.VMEM((L,), jnp.int32) for _ in range(2 * RING)]
      + [pltpu.VMEM_SHARED((NPAD, D), jnp.float32)]
      + [pltpu.SemaphoreType.DMA for _ in range(2 * RING)],
)
def _edge(h_hbm, p_hbm, q_hbm, src_hbm, dst_hbm, ew_hbm,
          agg_hbm, s_hbm,
          src_v, dst_v, ew_v, p_v, q_v, s_loc, tbuf, *rest):
    rows = list(rest[0:RING])
    ibufs = list(rest[RING:2 * RING])
    dbufs = list(rest[2 * RING:3 * RING])
    agg_s = rest[3 * RING]
    gsems = list(rest[3 * RING + 1:4 * RING + 1])
    ssems = list(rest[4 * RING + 1:5 * RING + 1])
    c = lax.axis_index("c")
    t = lax.axis_index("s")
    ebase = c * EPC + t * EPT
    pltpu.sync_copy(p_hbm, p_v)
    pltpu.sync_copy(q_hbm, q_v)

    zv = jnp.zeros((L,), jnp.float32)

    def _zs(i, carry):
        s_loc[pl.ds(i * L, L)] = zv
        return carry
    lax.fori_loop(0, NPAD // L, _zs, 0)

    for r in range(L):
        for k in range(D // L):
            rows[0][r, pl.ds(k * L, L)] = zv

    rbase = t * RPT

    def _za(j, carry):
        pltpu.sync_copy(rows[0], agg_s.at[pl.ds(rbase + j * L, L)])
        return carry
    lax.fori_loop(0, RPT // L, _za, 0)
    plsc.subcore_barrier()

    def _scale_scatter(rows, dbuf, ssem, tv, dv):
        dbuf[...] = dv
        for r in range(L):
            tr = tv[r]
            for k in range(D // L):
                rows[r, pl.ds(k * L, L)] = rows[r, pl.ds(k * L, L)] * tr
        pltpu.async_copy(rows, agg_s.at[dbuf], ssem, add=True)


    def _blk(bi, carry):
        eb = ebase + bi * EBLK
        pltpu.sync_copy(src_hbm.at[pl.ds(eb, EBLK)], src_v)
        pltpu.sync_copy(dst_hbm.at[pl.ds(eb, EBLK)], dst_v)
        pltpu.sync_copy(ew_hbm.at[pl.ds(eb, EBLK)], ew_v)

        for j in range(RING - 1):
            ibufs[j][...] = src_v[pl.ds(j * L, L)]
            pltpu.async_copy(h_hbm.at[ibufs[j]], rows[j], gsems[j])

        def _body(ci, carry2):
            par = lax.rem(ci, RING)
            off = ci * L
            sv = src_v[pl.ds(off, L)]
            dv = dst_v[pl.ds(off, L)]

            @pl.when(ci < NCHUNK - (RING - 1))
            def _():
                nsv = src_v[pl.ds(off + (RING - 1) * L, L)]
                npar = lax.rem(ci + RING - 1, RING)
                for j in range(RING):
                    @pl.when(npar == j)
                    def _(j=j):
                        @pl.when(ci >= 1)
                        def _():
                            pltpu.make_async_copy(
                                rows[j], agg_s.at[dbufs[j]], ssems[j]).wait()
                        ibufs[j][...] = nsv
                        pltpu.async_copy(h_hbm.at[ibufs[j]], rows[j],
                                         gsems[j])

            pj = plsc.load_gather(p_v, [dv])
            qj = plsc.load_gather(q_v, [sv])
            a = pj + qj
            a = jnp.where(a > 0, a, 0.2 * a)
            e = jnp.exp(a)
            plsc.addupdate_scatter(s_loc, [dv], e)
            tbuf[...] = e * ew_v[pl.ds(off, L)]
            tv = tbuf[...]

            for j in range(RING):
                @pl.when(par == j)
                def _(j=j):
                    pltpu.make_async_copy(
                        h_hbm.at[ibufs[j]], rows[j], gsems[j]).wait()
                    _scale_scatter(rows[j], dbufs[j], ssems[j], tv, dv)
            return carry2
        lax.fori_loop(0, NCHUNK, _body, 0)
        for j in range(RING):
            pltpu.make_async_copy(rows[j], agg_s.at[dbufs[j]],
                                  ssems[j]).wait()
        return carry
    lax.fori_loop(0, NBLK, _blk, 0)
    plsc.subcore_barrier()

    pltpu.sync_copy(s_loc, s_hbm.at[c, t])
    pltpu.sync_copy(agg_s.at[pl.ds(rbase, RPT)],
                    agg_hbm.at[c, pl.ds(rbase, RPT)])


def _ln_attn_body(h_ref, ls_ref, lb_ref, awd_ref, aws_ref,
                  hln_ref, p_ref, q_ref):
    xv = h_ref[...]
    mu = jnp.mean(xv, axis=-1, keepdims=True)
    var = jnp.mean(jnp.square(xv - mu), axis=-1, keepdims=True)
    hln = (xv - mu) / jnp.sqrt(var + 1e-5) * ls_ref[...] + lb_ref[...]
    hln_ref[...] = hln
    p_ref[...] = jnp.sum(hln * awd_ref[...], axis=-1, keepdims=True)
    q_ref[...] = jnp.sum(hln * aws_ref[...], axis=-1, keepdims=True)


def _ln_attn(h, ls, lb, awd, aws):
    row = pl.BlockSpec((BLK, D), lambda i: (i, 0))
    vec = pl.BlockSpec((1, D), lambda i: (0, 0))
    col = pl.BlockSpec((BLK, 1), lambda i: (i, 0))
    return pl.pallas_call(
        _ln_attn_body,
        grid=(GRID,),
        in_specs=[row, vec, vec, vec, vec],
        out_specs=[row, col, col],
        out_shape=[
            jax.ShapeDtypeStruct((NPAD, D), jnp.float32),
            jax.ShapeDtypeStruct((NPAD, 1), jnp.float32),
            jax.ShapeDtypeStruct((NPAD, 1), jnp.float32),
        ],
    )(h, ls, lb, awd, aws)


def _make_mlp_body(do_ln):
    def body(h_ref, a0_ref, a1_ref, s_ref, p_ref, q_ref, ep_ref,
             w1_ref, b1_ref, w2_ref, b2_ref, *rest):
        hln = h_ref[...]
        es = p_ref[...] + q_ref[...]
        es = jnp.where(es > 0, es, 0.2 * es)
        es = jnp.exp(es)
        stot = jnp.sum(s_ref[...], axis=-1, keepdims=True) + es
        aggf = (a0_ref[...] + a1_ref[...] + es * hln) / stot
        outv = ep_ref[...][:1, :1] * hln + aggf
        h1 = jnp.dot(outv, w1_ref[...],
                     preferred_element_type=jnp.float32) + b1_ref[...]
        h1 = jnp.maximum(h1, 0.0)
        y = jnp.dot(h1, w2_ref[...],
                    preferred_element_type=jnp.float32) + b2_ref[...]
        y = jnp.maximum(y, 0.0)
        if do_ln:
            ls_ref, lb_ref, awd_ref, aws_ref, hln_ref, p2_ref, q2_ref = rest
            mu = jnp.mean(y, axis=-1, keepdims=True)
            var = jnp.mean(jnp.square(y - mu), axis=-1, keepdims=True)
            y2 = (y - mu) / jnp.sqrt(var + 1e-5) * ls_ref[...] + lb_ref[...]
            hln_ref[...] = y2
            p2_ref[...] = jnp.sum(y2 * awd_ref[...], axis=-1, keepdims=True)
            q2_ref[...] = jnp.sum(y2 * aws_ref[...], axis=-1, keepdims=True)
        else:
            (y_ref,) = rest
            y_ref[...] = y
    return body


def _mlp(do_ln, h, a0, a1, st, p, q, ep, w1, b1, w2, b2, extra):
    row = pl.BlockSpec((BLK, D), lambda i: (i, 0))
    vec = pl.BlockSpec((1, D), lambda i: (0, 0))
    col = pl.BlockSpec((BLK, 1), lambda i: (i, 0))
    mat = pl.BlockSpec((D, D), lambda i: (0, 0))
    srow = pl.BlockSpec((BLK, NC * NS), lambda i: (i, 0))
    in_specs = [row, row, row, srow, col, col, vec, mat, vec, mat, vec]
    args = [h, a0, a1, st, p, q, ep, w1, b1, w2, b2]
    if do_ln:
        in_specs += [vec, vec, vec, vec]
        out_specs = [row, col, col]
        out_shape = [
            jax.ShapeDtypeStruct((NPAD, D), jnp.float32),
            jax.ShapeDtypeStruct((NPAD, 1), jnp.float32),
            jax.ShapeDtypeStruct((NPAD, 1), jnp.float32),
        ]
    else:
        out_specs = [row]
        out_shape = [jax.ShapeDtypeStruct((NPAD, D), jnp.float32)]
    return pl.pallas_call(
        _make_mlp_body(do_ln),
        grid=(GRID,),
        in_specs=in_specs,
        out_specs=out_specs,
        out_shape=out_shape,
    )(*args, *extra)


def _pool_body(y_ref, b_ref, w3_ref, b3_ref, w4_ref, b4_ref,
               out_ref, sums, cnts):
    i = pl.program_id(0)

    @pl.when(i == 0)
    def _():
        sums[...] = jnp.zeros((G, D), jnp.float32)
        cnts[...] = jnp.zeros((G, D), jnp.float32)

    bb = b_ref[...]
    oh = (lax.broadcasted_iota(jnp.int32, (G, BLK), 0) == bb
          ).astype(jnp.float32)
    sums[...] += jnp.dot(oh, y_ref[...], preferred_element_type=jnp.float32)
    cnts[...] += jnp.broadcast_to(
        jnp.sum(oh, axis=-1, keepdims=True), (G, D))

    @pl.when(i == GRID - 1)
    def _():
        g = sums[...] / jnp.maximum(cnts[...], 1.0)
        z = jnp.maximum(
            jnp.dot(g, w3_ref[...], preferred_element_type=jnp.float32)
            + b3_ref[...], 0.0)
        out_ref[...] = jnp.dot(
            z, w4_ref[...], preferred_element_type=jnp.float32) + b4_ref[...]


def _pool(y, bpad, w3, b3, w4, b4):
    return pl.pallas_call(
        _pool_body,
        grid=(GRID,),
        in_specs=[
            pl.BlockSpec((BLK, D), lambda i: (i, 0)),
            pl.BlockSpec((1, BLK), lambda i: (0, i)),
            pl.BlockSpec((D, G), lambda i: (0, 0)),
            pl.BlockSpec((1, G), lambda i: (0, 0)),
            pl.BlockSpec((G, 2), lambda i: (0, 0)),
            pl.BlockSpec((1, 2), lambda i: (0, 0)),
        ],
        out_specs=pl.BlockSpec((G, 2), lambda i: (0, 0)),
        out_shape=jax.ShapeDtypeStruct((G, 2), jnp.float32),
        scratch_shapes=[
            pltpu.VMEM((G, D), jnp.float32),
            pltpu.VMEM((G, D), jnp.float32),
        ],
        compiler_params=pltpu.CompilerParams(
            dimension_semantics=("arbitrary",)),
    )(y, bpad, w3, b3, w4, b4)


def kernel(x, edge_index, edge_attr, batch, emb,
           ln0_s, ln0_b, att0, w1_0, b1_0, w2_0, b2_0, eps0,
           ln1_s, ln1_b, att1, w1_1, b1_1, w2_1, b2_1, eps1,
           w3, b3, w4, b4):
    f32 = jnp.float32
    xi = jnp.concatenate([x.astype(jnp.int32),
                          jnp.zeros((NPAD - N,), jnp.int32)])
    src = edge_index[0].astype(jnp.int32)
    dst = edge_index[1].astype(jnp.int32)
    ew = edge_attr.astype(f32)
    bpad = jnp.concatenate([batch.astype(jnp.int32),
                            jnp.full((NPAD - N,), G, jnp.int32)]
                           ).reshape(1, NPAD)

    awd0 = att0[:D, 0].reshape(1, D)
    aws0 = att0[D:, 0].reshape(1, D)
    awd1 = att1[:D, 0].reshape(1, D)
    aws1 = att1[D:, 0].reshape(1, D)

    h0 = _emb_gather(emb.astype(f32), xi)
    hln1, p1, q1 = _ln_attn(h0, ln0_s.reshape(1, D), ln0_b.reshape(1, D),
                            awd0, aws0)
    agg1, s1 = _edge(hln1, p1.reshape(NPAD), q1.reshape(NPAD), src, dst, ew)
    s1t = jnp.swapaxes(s1.reshape(NC * NS, NPAD), 0, 1)
    ep0 = jnp.broadcast_to(jnp.reshape(1.0 + eps0, (1, 1)), (1, D))
    hln2, p2, q2 = _mlp(True, hln1, agg1[0], agg1[1], s1t, p1, q1, ep0,
                        w1_0, b1_0.reshape(1, D), w2_0, b2_0.reshape(1, D),
                        [ln1_s.reshape(1, D), ln1_b.reshape(1, D),
                         awd1, aws1])
    agg2, s2 = _edge(hln2, p2.reshape(NPAD), q2.reshape(NPAD), src, dst, ew)
    s2t = jnp.swapaxes(s2.reshape(NC * NS, NPAD), 0, 1)
    ep1 = jnp.broadcast_to(jnp.reshape(1.0 + eps1, (1, 1)), (1, D))
    (y2,) = _mlp(False, hln2, agg2[0], agg2[1], s2t, p2, q2, ep1,
                 w1_1, b1_1.reshape(1, D), w2_1, b2_1.reshape(1, D), [])
    return _pool(y2, bpad, w3, b3.reshape(1, G), w4, b4.reshape(1, 2))
